# baseline (speedup 1.0000x reference)
import dataclasses
import functools

import jax
import jax.numpy as jnp
from jax import lax
from jax.experimental import pallas as pl
from jax.experimental.pallas import tpu as pltpu
from jax.experimental.pallas import tpu_sc as plsc

N = 10000
E = 320000
G = 64
D = 128

NPAD = 10240
WE = 80
RPT = 128
NCHUNK = 8
CHW = RPT // NCHUNK
NB = 4
EPAD = 32 * RPT * WE
RB = 1000


def _vsc_mesh():
    return plsc.VectorSubcoreMesh(core_axis_name="c", subcore_axis_name="s")


def _sc_params():
    return dataclasses.replace(pltpu.CompilerParams(),
                               needs_layout_passes=False)


def _sc_degree(col16):
    NV = (E // 16) // 32
    STRIDE = NPAD // 16

    @functools.partial(
        pl.kernel,
        out_type=jax.ShapeDtypeStruct((2, NPAD), jnp.float32),
        mesh=_vsc_mesh(),
        scratch_types=[
            pltpu.VMEM((NV, 16), jnp.int32),
            pltpu.VMEM((NPAD,), jnp.float32),
            pltpu.VMEM((16, STRIDE), jnp.float32),
            pltpu.VMEM((STRIDE,), jnp.float32),
            pltpu.VMEM_SHARED((16, NPAD), jnp.float32),
        ],
        compiler_params=_sc_params(),
    )
    def k(col_hbm, out_hbm, idx_v, hist_v, rbuf, rout, hists_sh):
        c = lax.axis_index("c")
        s = lax.axis_index("s")
        wid = s * 2 + c

        @pl.loop(0, NPAD // 16)
        def _(i):
            hist_v[pl.ds(i * 16, 16)] = jnp.zeros((16,), jnp.float32)

        pltpu.sync_copy(col_hbm.at[wid], idx_v)

        @pl.loop(0, NV)
        def _(j):
            v = idx_v[j, :]
            vals, msk = plsc.scan_count(v)
            plsc.addupdate_scatter(hist_v, [v], vals.astype(jnp.float32),
                                   mask=msk)

        pltpu.sync_copy(hist_v, hists_sh.at[s])
        plsc.subcore_barrier()

        for t in range(16):
            pltpu.sync_copy(hists_sh.at[t, pl.ds(s * STRIDE, STRIDE)],
                            rbuf.at[t])

        @pl.loop(0, STRIDE // 16)
        def _(kk):
            a = rbuf[0, pl.ds(kk * 16, 16)]
            for t in range(1, 16):
                a = a + rbuf[t, pl.ds(kk * 16, 16)]
            rout[pl.ds(kk * 16, 16)] = a

        pltpu.sync_copy(rout, out_hbm.at[c, pl.ds(s * STRIDE, STRIDE)])

    return k(col16)


def _sc_scatter(y, row2d, col2d):
    @functools.partial(
        pl.kernel,
        out_type=jax.ShapeDtypeStruct((2, NPAD, D), jnp.float32),
        mesh=_vsc_mesh(),
        scratch_types=[
            pltpu.VMEM((CHW, WE), jnp.int32),
            pltpu.VMEM((CHW, WE), jnp.int32),
        ] + [pltpu.VMEM((WE, D), jnp.float32)] * NB + [
            pltpu.VMEM_SHARED((NPAD, D), jnp.float32),
        ] + [pltpu.SemaphoreType.DMA] * NB,
    )
    def k(y_hbm, row_hbm, col_hbm, out_hbm, row_v, col_v, *rest):
        gbufs = rest[:NB]
        accum = rest[NB]
        gsems = rest[NB + 1:]
        c = lax.axis_index("c")
        s = lax.axis_index("s")
        wid = s * 2 + c

        @pl.loop(0, 16)
        def _(i):
            @pl.loop(0, D // 16)
            def _(j):
                gbufs[0][i, pl.ds(j * 16, 16)] = jnp.zeros((16,), jnp.float32)

        @pl.loop(0, 40)
        def _(i):
            pltpu.sync_copy(gbufs[0].at[pl.ds(0, 16)],
                            accum.at[pl.ds(s * 640 + i * 16, 16)])

        plsc.subcore_barrier()

        @pl.loop(0, NCHUNK)
        def _(h):
            pltpu.sync_copy(row_hbm.at[wid, h], row_v)
            pltpu.sync_copy(col_hbm.at[wid, h], col_v)
            for b in range(NB - 1):
                pltpu.async_copy(y_hbm.at[row_v.at[b]], gbufs[b], gsems[b])

            @pl.loop(0, CHW // NB)
            def _(i):
                j0 = NB * i
                for b in range(NB):
                    pltpu.make_async_copy(y_hbm.at[row_v.at[j0 + b]],
                                          gbufs[b], gsems[b]).wait()
                    jn = j0 + b + NB - 1
                    bn = (b + NB - 1) % NB

                    @pl.when(jn < CHW)
                    def _(jn=jn, bn=bn):
                        pltpu.async_copy(y_hbm.at[row_v.at[jn]], gbufs[bn],
                                         gsems[bn])

                    pltpu.sync_copy(gbufs[b], accum.at[col_v.at[j0 + b]],
                                    add=True)

        plsc.subcore_barrier()
        pltpu.sync_copy(accum.at[pl.ds(s * 640, 640)],
                        out_hbm.at[c, pl.ds(s * 640, 640)])

    return k(y, row2d, col2d)


def _mm_scale_kernel(x_ref, w_ref, d0_ref, d1_ref, o_ref):
    dis = lax.rsqrt(1.0 + d0_ref[...] + d1_ref[...])
    h = jnp.dot(x_ref[...], w_ref[...], preferred_element_type=jnp.float32)
    o_ref[...] = h * dis


def _mm_scale(x, W, d0, d1):
    return pl.pallas_call(
        _mm_scale_kernel,
        grid=(N // RB,),
        in_specs=[pl.BlockSpec((RB, D), lambda i: (i, 0)),
                  pl.BlockSpec((D, D), lambda i: (0, 0)),
                  pl.BlockSpec((RB, 1), lambda i: (i, 0)),
                  pl.BlockSpec((RB, 1), lambda i: (i, 0))],
        out_specs=pl.BlockSpec((RB, D), lambda i: (i, 0)),
        out_shape=jax.ShapeDtypeStruct((N, D), jnp.float32),
    )(x, W, d0, d1)


def _layer_kernel(p0_ref, p1_ref, y_ref, d0_ref, d1_ref, b_ref, w_ref, o_ref):
    dis = lax.rsqrt(1.0 + d0_ref[...] + d1_ref[...])
    t = (p0_ref[...] + p1_ref[...] + y_ref[...]) * dis + b_ref[...]
    t = jnp.maximum(t, 0.0)
    h = jnp.dot(t, w_ref[...], preferred_element_type=jnp.float32)
    o_ref[...] = h * dis


def _layer(p0, p1, y, d0, d1, b, W):
    return pl.pallas_call(
        _layer_kernel,
        grid=(N // RB,),
        in_specs=[pl.BlockSpec((RB, D), lambda i: (i, 0)),
                  pl.BlockSpec((RB, D), lambda i: (i, 0)),
                  pl.BlockSpec((RB, D), lambda i: (i, 0)),
                  pl.BlockSpec((RB, 1), lambda i: (i, 0)),
                  pl.BlockSpec((RB, 1), lambda i: (i, 0)),
                  pl.BlockSpec((1, D), lambda i: (0, 0)),
                  pl.BlockSpec((D, D), lambda i: (0, 0))],
        out_specs=pl.BlockSpec((RB, D), lambda i: (i, 0)),
        out_shape=jax.ShapeDtypeStruct((N, D), jnp.float32),
    )(p0, p1, y, d0, d1, b.reshape(1, D), W)


def _pool_kernel(p0_ref, p1_ref, y_ref, d0_ref, d1_ref, b_ref, batch_ref,
                 o_ref, acc, cnt):
    i = pl.program_id(0)

    @pl.when(i == 0)
    def _():
        acc[...] = jnp.zeros_like(acc)
        cnt[...] = jnp.zeros_like(cnt)

    dis = lax.rsqrt(1.0 + d0_ref[...] + d1_ref[...])
    h = (p0_ref[...] + p1_ref[...] + y_ref[...]) * dis + b_ref[...]
    gid = lax.broadcasted_iota(jnp.int32, (1, G), 1).astype(jnp.float32)
    sel = (batch_ref[...] == gid).astype(jnp.float32)
    acc[...] += lax.dot_general(sel, h, (((0,), (0,)), ((), ())),
                                preferred_element_type=jnp.float32)
    cnt[...] += lax.dot_general(sel, jnp.ones_like(h), (((0,), (0,)), ((), ())),
                                preferred_element_type=jnp.float32)

    @pl.when(i == pl.num_programs(0) - 1)
    def _():
        o_ref[...] = acc[...] / jnp.maximum(cnt[...], 1.0)


def _pool(p0, p1, y, d0, d1, b, batchf):
    return pl.pallas_call(
        _pool_kernel,
        grid=(N // RB,),
        in_specs=[pl.BlockSpec((RB, D), lambda i: (i, 0)),
                  pl.BlockSpec((RB, D), lambda i: (i, 0)),
                  pl.BlockSpec((RB, D), lambda i: (i, 0)),
                  pl.BlockSpec((RB, 1), lambda i: (i, 0)),
                  pl.BlockSpec((RB, 1), lambda i: (i, 0)),
                  pl.BlockSpec((1, D), lambda i: (0, 0)),
                  pl.BlockSpec((RB, 1), lambda i: (i, 0))],
        out_specs=pl.BlockSpec((G, D), lambda i: (0, 0)),
        out_shape=jax.ShapeDtypeStruct((G, D), jnp.float32),
        scratch_shapes=[pltpu.VMEM((G, D), jnp.float32),
                        pltpu.VMEM((G, D), jnp.float32)],
    )(p0, p1, y, d0, d1, b.reshape(1, D), batchf)


def kernel(x, W1, b1, W2, b2, W3, b3, edge_index, batch):
    x = x.astype(jnp.float32)
    npad_e = EPAD - E
    pad_row = (jnp.arange(npad_e, dtype=jnp.int32) % 1024)
    pad_col = N + (jnp.arange(npad_e, dtype=jnp.int32) % (NPAD - N))
    row2d = jnp.concatenate([edge_index[0], pad_row]).reshape(32, NCHUNK,
                                                              CHW, WE)
    col2d = jnp.concatenate([edge_index[1], pad_col]).reshape(32, NCHUNK,
                                                              CHW, WE)
    col16 = edge_index[1].reshape(32, E // (16 * 32), 16)
    batchf = batch.astype(jnp.float32).reshape(N, 1)

    dsum = _sc_degree(col16)
    d0 = dsum[0, :N].reshape(N, 1)
    d1 = dsum[1, :N].reshape(N, 1)

    y1 = _mm_scale(x, W1, d0, d1)
    p = _sc_scatter(y1, row2d, col2d)
    y2 = _layer(p[0, :N], p[1, :N], y1, d0, d1, b1, W2)
    q = _sc_scatter(y2, row2d, col2d)
    y3 = _layer(q[0, :N], q[1, :N], y2, d0, d1, b2, W3)
    r = _sc_scatter(y3, row2d, col2d)
    return _pool(r[0, :N], r[1, :N], y3, d0, d1, b3, batchf)


# speedup vs baseline: 1.0430x; 1.0430x over previous
import dataclasses
import functools

import jax
import jax.numpy as jnp
from jax import lax
from jax.experimental import pallas as pl
from jax.experimental.pallas import tpu as pltpu
from jax.experimental.pallas import tpu_sc as plsc

N = 10000
E = 320000
G = 64
D = 128

NPAD = 10240
WE = 112
RPT = 90
NCHUNK = 6
CHW = RPT // NCHUNK
NB = 3
EPAD = 32 * RPT * WE
RB = 1000


def _vsc_mesh():
    return plsc.VectorSubcoreMesh(core_axis_name="c", subcore_axis_name="s")


def _sc_params():
    return dataclasses.replace(pltpu.CompilerParams(),
                               needs_layout_passes=False)


def _sc_degree(col16):
    NV = (E // 16) // 32
    STRIDE = NPAD // 16

    @functools.partial(
        pl.kernel,
        out_type=jax.ShapeDtypeStruct((2, NPAD), jnp.float32),
        mesh=_vsc_mesh(),
        scratch_types=[
            pltpu.VMEM((NV, 16), jnp.int32),
            pltpu.VMEM((NPAD,), jnp.float32),
            pltpu.VMEM((16, STRIDE), jnp.float32),
            pltpu.VMEM((STRIDE,), jnp.float32),
            pltpu.VMEM_SHARED((16, NPAD), jnp.float32),
        ],
        compiler_params=_sc_params(),
    )
    def k(col_hbm, out_hbm, idx_v, hist_v, rbuf, rout, hists_sh):
        c = lax.axis_index("c")
        s = lax.axis_index("s")
        wid = s * 2 + c

        @pl.loop(0, NPAD // 16)
        def _(i):
            hist_v[pl.ds(i * 16, 16)] = jnp.zeros((16,), jnp.float32)

        pltpu.sync_copy(col_hbm.at[wid], idx_v)

        @pl.loop(0, NV)
        def _(j):
            v = idx_v[j, :]
            vals, msk = plsc.scan_count(v)
            plsc.addupdate_scatter(hist_v, [v], vals.astype(jnp.float32),
                                   mask=msk)

        pltpu.sync_copy(hist_v, hists_sh.at[s])
        plsc.subcore_barrier()

        for t in range(16):
            pltpu.sync_copy(hists_sh.at[t, pl.ds(s * STRIDE, STRIDE)],
                            rbuf.at[t])

        @pl.loop(0, STRIDE // 16)
        def _(kk):
            a = rbuf[0, pl.ds(kk * 16, 16)]
            for t in range(1, 16):
                a = a + rbuf[t, pl.ds(kk * 16, 16)]
            rout[pl.ds(kk * 16, 16)] = a

        pltpu.sync_copy(rout, out_hbm.at[c, pl.ds(s * STRIDE, STRIDE)])

    return k(col16)


def _sc_scatter(y, row2d, col2d):
    @functools.partial(
        pl.kernel,
        out_type=jax.ShapeDtypeStruct((2, NPAD, D), jnp.float32),
        mesh=_vsc_mesh(),
        scratch_types=[
            pltpu.VMEM((CHW, WE), jnp.int32),
            pltpu.VMEM((CHW, WE), jnp.int32),
        ] + [pltpu.VMEM((WE, D), jnp.float32)] * NB + [
            pltpu.VMEM_SHARED((NPAD, D), jnp.float32),
        ] + [pltpu.SemaphoreType.DMA] * NB,
    )
    def k(y_hbm, row_hbm, col_hbm, out_hbm, row_v, col_v, *rest):
        gbufs = rest[:NB]
        accum = rest[NB]
        gsems = rest[NB + 1:]
        c = lax.axis_index("c")
        s = lax.axis_index("s")
        wid = s * 2 + c

        @pl.loop(0, 16)
        def _(i):
            @pl.loop(0, D // 16)
            def _(j):
                gbufs[0][i, pl.ds(j * 16, 16)] = jnp.zeros((16,), jnp.float32)

        @pl.loop(0, 40)
        def _(i):
            pltpu.sync_copy(gbufs[0].at[pl.ds(0, 16)],
                            accum.at[pl.ds(s * 640 + i * 16, 16)])

        plsc.subcore_barrier()

        @pl.loop(0, NCHUNK)
        def _(h):
            pltpu.sync_copy(row_hbm.at[wid, h], row_v)
            pltpu.sync_copy(col_hbm.at[wid, h], col_v)
            for b in range(NB - 1):
                pltpu.async_copy(y_hbm.at[row_v.at[b]], gbufs[b], gsems[b])

            @pl.loop(0, CHW // NB)
            def _(i):
                j0 = NB * i
                for b in range(NB):
                    pltpu.make_async_copy(y_hbm.at[row_v.at[j0 + b]],
                                          gbufs[b], gsems[b]).wait()
                    jn = j0 + b + NB - 1
                    bn = (b + NB - 1) % NB

                    @pl.when(jn < CHW)
                    def _(jn=jn, bn=bn):
                        pltpu.async_copy(y_hbm.at[row_v.at[jn]], gbufs[bn],
                                         gsems[bn])

                    pltpu.sync_copy(gbufs[b], accum.at[col_v.at[j0 + b]],
                                    add=True)

        plsc.subcore_barrier()
        pltpu.sync_copy(accum.at[pl.ds(s * 640, 640)],
                        out_hbm.at[c, pl.ds(s * 640, 640)])

    return k(y, row2d, col2d)


def _mm_scale_kernel(x_ref, w_ref, d0_ref, d1_ref, o_ref):
    dis = lax.rsqrt(1.0 + d0_ref[...] + d1_ref[...])
    h = jnp.dot(x_ref[...], w_ref[...], preferred_element_type=jnp.float32)
    o_ref[...] = h * dis


def _mm_scale(x, W, d0, d1):
    return pl.pallas_call(
        _mm_scale_kernel,
        grid=(N // RB,),
        in_specs=[pl.BlockSpec((RB, D), lambda i: (i, 0)),
                  pl.BlockSpec((D, D), lambda i: (0, 0)),
                  pl.BlockSpec((RB, 1), lambda i: (i, 0)),
                  pl.BlockSpec((RB, 1), lambda i: (i, 0))],
        out_specs=pl.BlockSpec((RB, D), lambda i: (i, 0)),
        out_shape=jax.ShapeDtypeStruct((N, D), jnp.float32),
    )(x, W, d0, d1)


def _layer_kernel(p0_ref, p1_ref, y_ref, d0_ref, d1_ref, b_ref, w_ref, o_ref):
    dis = lax.rsqrt(1.0 + d0_ref[...] + d1_ref[...])
    t = (p0_ref[...] + p1_ref[...] + y_ref[...]) * dis + b_ref[...]
    t = jnp.maximum(t, 0.0)
    h = jnp.dot(t, w_ref[...], preferred_element_type=jnp.float32)
    o_ref[...] = h * dis


def _layer(p0, p1, y, d0, d1, b, W):
    return pl.pallas_call(
        _layer_kernel,
        grid=(N // RB,),
        in_specs=[pl.BlockSpec((RB, D), lambda i: (i, 0)),
                  pl.BlockSpec((RB, D), lambda i: (i, 0)),
                  pl.BlockSpec((RB, D), lambda i: (i, 0)),
                  pl.BlockSpec((RB, 1), lambda i: (i, 0)),
                  pl.BlockSpec((RB, 1), lambda i: (i, 0)),
                  pl.BlockSpec((1, D), lambda i: (0, 0)),
                  pl.BlockSpec((D, D), lambda i: (0, 0))],
        out_specs=pl.BlockSpec((RB, D), lambda i: (i, 0)),
        out_shape=jax.ShapeDtypeStruct((N, D), jnp.float32),
    )(p0, p1, y, d0, d1, b.reshape(1, D), W)


def _pool_kernel(p0_ref, p1_ref, y_ref, d0_ref, d1_ref, b_ref, batch_ref,
                 o_ref, acc, cnt):
    i = pl.program_id(0)

    @pl.when(i == 0)
    def _():
        acc[...] = jnp.zeros_like(acc)
        cnt[...] = jnp.zeros_like(cnt)

    dis = lax.rsqrt(1.0 + d0_ref[...] + d1_ref[...])
    h = (p0_ref[...] + p1_ref[...] + y_ref[...]) * dis + b_ref[...]
    gid = lax.broadcasted_iota(jnp.int32, (1, G), 1).astype(jnp.float32)
    sel = (batch_ref[...] == gid).astype(jnp.float32)
    acc[...] += lax.dot_general(sel, h, (((0,), (0,)), ((), ())),
                                preferred_element_type=jnp.float32)
    cnt[...] += lax.dot_general(sel, jnp.ones_like(h), (((0,), (0,)), ((), ())),
                                preferred_element_type=jnp.float32)

    @pl.when(i == pl.num_programs(0) - 1)
    def _():
        o_ref[...] = acc[...] / jnp.maximum(cnt[...], 1.0)


def _pool(p0, p1, y, d0, d1, b, batchf):
    return pl.pallas_call(
        _pool_kernel,
        grid=(N // RB,),
        in_specs=[pl.BlockSpec((RB, D), lambda i: (i, 0)),
                  pl.BlockSpec((RB, D), lambda i: (i, 0)),
                  pl.BlockSpec((RB, D), lambda i: (i, 0)),
                  pl.BlockSpec((RB, 1), lambda i: (i, 0)),
                  pl.BlockSpec((RB, 1), lambda i: (i, 0)),
                  pl.BlockSpec((1, D), lambda i: (0, 0)),
                  pl.BlockSpec((RB, 1), lambda i: (i, 0))],
        out_specs=pl.BlockSpec((G, D), lambda i: (0, 0)),
        out_shape=jax.ShapeDtypeStruct((G, D), jnp.float32),
        scratch_shapes=[pltpu.VMEM((G, D), jnp.float32),
                        pltpu.VMEM((G, D), jnp.float32)],
    )(p0, p1, y, d0, d1, b.reshape(1, D), batchf)


def kernel(x, W1, b1, W2, b2, W3, b3, edge_index, batch):
    x = x.astype(jnp.float32)
    npad_e = EPAD - E
    pad_row = (jnp.arange(npad_e, dtype=jnp.int32) % 1024)
    pad_col = N + (jnp.arange(npad_e, dtype=jnp.int32) % (NPAD - N))
    row2d = jnp.concatenate([edge_index[0], pad_row]).reshape(32, NCHUNK,
                                                              CHW, WE)
    col2d = jnp.concatenate([edge_index[1], pad_col]).reshape(32, NCHUNK,
                                                              CHW, WE)
    col16 = edge_index[1].reshape(32, E // (16 * 32), 16)
    batchf = batch.astype(jnp.float32).reshape(N, 1)

    dsum = _sc_degree(col16)
    d0 = dsum[0, :N].reshape(N, 1)
    d1 = dsum[1, :N].reshape(N, 1)

    y1 = _mm_scale(x, W1, d0, d1)
    p = _sc_scatter(y1, row2d, col2d)
    y2 = _layer(p[0, :N], p[1, :N], y1, d0, d1, b1, W2)
    q = _sc_scatter(y2, row2d, col2d)
    y3 = _layer(q[0, :N], q[1, :N], y2, d0, d1, b2, W3)
    r = _sc_scatter(y3, row2d, col2d)
    return _pool(r[0, :N], r[1, :N], y3, d0, d1, b3, batchf)


# speedup vs baseline: 1.0599x; 1.0162x over previous
import dataclasses
import functools

import jax
import jax.numpy as jnp
from jax import lax
from jax.experimental import pallas as pl
from jax.experimental.pallas import tpu as pltpu
from jax.experimental.pallas import tpu_sc as plsc

N = 10000
E = 320000
G = 64
D = 128

NPAD = 10240
WE = 112
RPT = 90
NCHUNK = 5
CHW = RPT // NCHUNK
NB = 3
EPAD = 32 * RPT * WE
RB = 1000


def _vsc_mesh():
    return plsc.VectorSubcoreMesh(core_axis_name="c", subcore_axis_name="s")


def _sc_params():
    return dataclasses.replace(pltpu.CompilerParams(),
                               needs_layout_passes=False)


def _sc_degree(col16):
    NV = (E // 16) // 32
    STRIDE = NPAD // 16

    @functools.partial(
        pl.kernel,
        out_type=jax.ShapeDtypeStruct((2, NPAD), jnp.float32),
        mesh=_vsc_mesh(),
        scratch_types=[
            pltpu.VMEM((NV, 16), jnp.int32),
            pltpu.VMEM((NPAD,), jnp.float32),
            pltpu.VMEM((16, STRIDE), jnp.float32),
            pltpu.VMEM((STRIDE,), jnp.float32),
            pltpu.VMEM_SHARED((16, NPAD), jnp.float32),
        ],
        compiler_params=_sc_params(),
    )
    def k(col_hbm, out_hbm, idx_v, hist_v, rbuf, rout, hists_sh):
        c = lax.axis_index("c")
        s = lax.axis_index("s")
        wid = s * 2 + c

        @pl.loop(0, NPAD // 16)
        def _(i):
            hist_v[pl.ds(i * 16, 16)] = jnp.zeros((16,), jnp.float32)

        pltpu.sync_copy(col_hbm.at[wid], idx_v)

        @pl.loop(0, NV)
        def _(j):
            v = idx_v[j, :]
            vals, msk = plsc.scan_count(v)
            plsc.addupdate_scatter(hist_v, [v], vals.astype(jnp.float32),
                                   mask=msk)

        pltpu.sync_copy(hist_v, hists_sh.at[s])
        plsc.subcore_barrier()

        for t in range(16):
            pltpu.sync_copy(hists_sh.at[t, pl.ds(s * STRIDE, STRIDE)],
                            rbuf.at[t])

        @pl.loop(0, STRIDE // 16)
        def _(kk):
            a = rbuf[0, pl.ds(kk * 16, 16)]
            for t in range(1, 16):
                a = a + rbuf[t, pl.ds(kk * 16, 16)]
            rout[pl.ds(kk * 16, 16)] = a

        pltpu.sync_copy(rout, out_hbm.at[c, pl.ds(s * STRIDE, STRIDE)])

    return k(col16)


def _sc_scatter(y, row2d, col2d):
    @functools.partial(
        pl.kernel,
        out_type=jax.ShapeDtypeStruct((2, NPAD, D), jnp.float32),
        mesh=_vsc_mesh(),
        scratch_types=[
            pltpu.VMEM((CHW, WE), jnp.int32),
            pltpu.VMEM((CHW, WE), jnp.int32),
        ] + [pltpu.VMEM((WE, D), jnp.float32)] * NB + [
            pltpu.VMEM_SHARED((NPAD, D), jnp.float32),
        ] + [pltpu.SemaphoreType.DMA] * NB,
    )
    def k(y_hbm, row_hbm, col_hbm, out_hbm, row_v, col_v, *rest):
        gbufs = rest[:NB]
        accum = rest[NB]
        gsems = rest[NB + 1:]
        c = lax.axis_index("c")
        s = lax.axis_index("s")
        wid = s * 2 + c

        @pl.loop(0, 16)
        def _(i):
            @pl.loop(0, D // 16)
            def _(j):
                gbufs[0][i, pl.ds(j * 16, 16)] = jnp.zeros((16,), jnp.float32)

        @pl.loop(0, 40)
        def _(i):
            pltpu.sync_copy(gbufs[0].at[pl.ds(0, 16)],
                            accum.at[pl.ds(s * 640 + i * 16, 16)])

        plsc.subcore_barrier()

        @pl.loop(0, NCHUNK)
        def _(h):
            pltpu.sync_copy(row_hbm.at[wid, h], row_v)
            pltpu.sync_copy(col_hbm.at[wid, h], col_v)
            for b in range(NB - 1):
                pltpu.async_copy(y_hbm.at[row_v.at[b]], gbufs[b], gsems[b])

            @pl.loop(0, CHW // NB)
            def _(i):
                j0 = NB * i
                for b in range(NB):
                    pltpu.make_async_copy(y_hbm.at[row_v.at[j0 + b]],
                                          gbufs[b], gsems[b]).wait()
                    jn = j0 + b + NB - 1
                    bn = (b + NB - 1) % NB

                    @pl.when(jn < CHW)
                    def _(jn=jn, bn=bn):
                        pltpu.async_copy(y_hbm.at[row_v.at[jn]], gbufs[bn],
                                         gsems[bn])

                    pltpu.sync_copy(gbufs[b], accum.at[col_v.at[j0 + b]],
                                    add=True)

        plsc.subcore_barrier()
        pltpu.sync_copy(accum.at[pl.ds(s * 640, 640)],
                        out_hbm.at[c, pl.ds(s * 640, 640)])

    return k(y, row2d, col2d)


def _mm_scale_kernel(x_ref, w_ref, d0_ref, d1_ref, o_ref):
    dis = lax.rsqrt(1.0 + d0_ref[...] + d1_ref[...])
    h = jnp.dot(x_ref[...], w_ref[...], preferred_element_type=jnp.float32)
    o_ref[...] = h * dis


def _mm_scale(x, W, d0, d1):
    return pl.pallas_call(
        _mm_scale_kernel,
        grid=(N // RB,),
        in_specs=[pl.BlockSpec((RB, D), lambda i: (i, 0)),
                  pl.BlockSpec((D, D), lambda i: (0, 0)),
                  pl.BlockSpec((RB, 1), lambda i: (i, 0)),
                  pl.BlockSpec((RB, 1), lambda i: (i, 0))],
        out_specs=pl.BlockSpec((RB, D), lambda i: (i, 0)),
        out_shape=jax.ShapeDtypeStruct((N, D), jnp.float32),
    )(x, W, d0, d1)


def _layer_kernel(p0_ref, p1_ref, y_ref, d0_ref, d1_ref, b_ref, w_ref, o_ref):
    dis = lax.rsqrt(1.0 + d0_ref[...] + d1_ref[...])
    t = (p0_ref[...] + p1_ref[...] + y_ref[...]) * dis + b_ref[...]
    t = jnp.maximum(t, 0.0)
    h = jnp.dot(t, w_ref[...], preferred_element_type=jnp.float32)
    o_ref[...] = h * dis


def _layer(p0, p1, y, d0, d1, b, W):
    return pl.pallas_call(
        _layer_kernel,
        grid=(N // RB,),
        in_specs=[pl.BlockSpec((RB, D), lambda i: (i, 0)),
                  pl.BlockSpec((RB, D), lambda i: (i, 0)),
                  pl.BlockSpec((RB, D), lambda i: (i, 0)),
                  pl.BlockSpec((RB, 1), lambda i: (i, 0)),
                  pl.BlockSpec((RB, 1), lambda i: (i, 0)),
                  pl.BlockSpec((1, D), lambda i: (0, 0)),
                  pl.BlockSpec((D, D), lambda i: (0, 0))],
        out_specs=pl.BlockSpec((RB, D), lambda i: (i, 0)),
        out_shape=jax.ShapeDtypeStruct((N, D), jnp.float32),
    )(p0, p1, y, d0, d1, b.reshape(1, D), W)


def _pool_kernel(p0_ref, p1_ref, y_ref, d0_ref, d1_ref, b_ref, batch_ref,
                 o_ref, acc, cnt):
    i = pl.program_id(0)

    @pl.when(i == 0)
    def _():
        acc[...] = jnp.zeros_like(acc)
        cnt[...] = jnp.zeros_like(cnt)

    dis = lax.rsqrt(1.0 + d0_ref[...] + d1_ref[...])
    h = (p0_ref[...] + p1_ref[...] + y_ref[...]) * dis + b_ref[...]
    gid = lax.broadcasted_iota(jnp.int32, (1, G), 1).astype(jnp.float32)
    sel = (batch_ref[...] == gid).astype(jnp.float32)
    acc[...] += lax.dot_general(sel, h, (((0,), (0,)), ((), ())),
                                preferred_element_type=jnp.float32)
    cnt[...] += lax.dot_general(sel, jnp.ones_like(h), (((0,), (0,)), ((), ())),
                                preferred_element_type=jnp.float32)

    @pl.when(i == pl.num_programs(0) - 1)
    def _():
        o_ref[...] = acc[...] / jnp.maximum(cnt[...], 1.0)


def _pool(p0, p1, y, d0, d1, b, batchf):
    return pl.pallas_call(
        _pool_kernel,
        grid=(N // RB,),
        in_specs=[pl.BlockSpec((RB, D), lambda i: (i, 0)),
                  pl.BlockSpec((RB, D), lambda i: (i, 0)),
                  pl.BlockSpec((RB, D), lambda i: (i, 0)),
                  pl.BlockSpec((RB, 1), lambda i: (i, 0)),
                  pl.BlockSpec((RB, 1), lambda i: (i, 0)),
                  pl.BlockSpec((1, D), lambda i: (0, 0)),
                  pl.BlockSpec((RB, 1), lambda i: (i, 0))],
        out_specs=pl.BlockSpec((G, D), lambda i: (0, 0)),
        out_shape=jax.ShapeDtypeStruct((G, D), jnp.float32),
        scratch_shapes=[pltpu.VMEM((G, D), jnp.float32),
                        pltpu.VMEM((G, D), jnp.float32)],
    )(p0, p1, y, d0, d1, b.reshape(1, D), batchf)


def kernel(x, W1, b1, W2, b2, W3, b3, edge_index, batch):
    x = x.astype(jnp.float32)
    npad_e = EPAD - E
    pad_row = (jnp.arange(npad_e, dtype=jnp.int32) % 1024)
    pad_col = N + (jnp.arange(npad_e, dtype=jnp.int32) % (NPAD - N))
    row2d = jnp.concatenate([edge_index[0], pad_row]).reshape(32, NCHUNK,
                                                              CHW, WE)
    col2d = jnp.concatenate([edge_index[1], pad_col]).reshape(32, NCHUNK,
                                                              CHW, WE)
    col16 = edge_index[1].reshape(32, E // (16 * 32), 16)
    batchf = batch.astype(jnp.float32).reshape(N, 1)

    dsum = _sc_degree(col16)
    d0 = dsum[0, :N].reshape(N, 1)
    d1 = dsum[1, :N].reshape(N, 1)

    y1 = _mm_scale(x, W1, d0, d1)
    p = _sc_scatter(y1, row2d, col2d)
    y2 = _layer(p[0, :N], p[1, :N], y1, d0, d1, b1, W2)
    q = _sc_scatter(y2, row2d, col2d)
    y3 = _layer(q[0, :N], q[1, :N], y2, d0, d1, b2, W3)
    r = _sc_scatter(y3, row2d, col2d)
    return _pool(r[0, :N], r[1, :N], y3, d0, d1, b3, batchf)
